# Initial kernel scaffold; baseline (speedup 1.0000x reference)
import jax
import jax.numpy as jnp
from jax.experimental import pallas as pl


def kernel(samples, tuple_mapping, ram_table):
    raise NotImplementedError("write your pallas kernel here")


# speedup vs baseline: 1.2707x; 1.2707x over previous
import functools

import jax
import jax.numpy as jnp
import numpy as np
from jax import lax
from jax.experimental import pallas as pl
from jax.experimental.pallas import tpu as pltpu
from jax.experimental.pallas import tpu_sc as plsc

_TUPLE = 16
_NR = 64
_NC = 10
_NP = _NC * _NR
_NA = 1 << _TUPLE
_B = 4096
_E = _NR * _TUPLE

_BN = 512
_NCORE = 2
_NSUB = 16
_CPC = _NC // _NCORE
_RPS = _NR // _NSUB
_LANES = 16


def _addr_body(map_ref, s_ref, out_ref, wt_ref):
    @pl.when(pl.program_id(0) == 0)
    def _():
        j = lax.broadcasted_iota(jnp.int32, (_NP, _E), 1)
        wt = jnp.zeros((_NP, _E), jnp.float32)
        for t in range(_TUPLE):
            col = map_ref[:, t : t + 1]
            wt = wt + jnp.where(col == j, np.float32(1 << (15 - t)),
                                np.float32(0.0))
        wt_ref[...] = wt.astype(jnp.bfloat16)

    acc = lax.dot_general(
        wt_ref[...], s_ref[...],
        dimension_numbers=(((1,), (0,)), ((), ())),
        preferred_element_type=jnp.float32)
    out_ref[...] = acc.astype(jnp.int32)


def _compute_addr(map2, samples_bf):
    return pl.pallas_call(
        _addr_body,
        grid=(_B // _BN,),
        in_specs=[
            pl.BlockSpec((_NP, _TUPLE), lambda i: (0, 0)),
            pl.BlockSpec((_E, _BN), lambda i: (0, i)),
        ],
        out_specs=pl.BlockSpec((_NP, _BN), lambda i: (0, i)),
        out_shape=jax.ShapeDtypeStruct((_NP, _B), jnp.int32),
        scratch_shapes=[pltpu.VMEM((_NP, _E), jnp.bfloat16)],
        compiler_params=pltpu.CompilerParams(
            dimension_semantics=("arbitrary",)),
    )(map2, samples_bf)


_sc_mesh = plsc.VectorSubcoreMesh(core_axis_name="c", subcore_axis_name="s")


@functools.partial(
    pl.kernel,
    out_type=jax.ShapeDtypeStruct((_NC * _B,), jnp.float32),
    mesh=_sc_mesh,
    scratch_types=[
        pltpu.VMEM((_NA,), jnp.float32),
        pltpu.VMEM((_B,), jnp.int32),
        pltpu.VMEM((_CPC * _B,), jnp.float32),
        pltpu.VMEM_SHARED((_CPC * _NSUB * _B,), jnp.float32),
    ],
    compiler_params=pltpu.CompilerParams(needs_layout_passes=False),
)
def _sc_lookup(table_hbm, addr_hbm, zeros_hbm, out_hbm,
               row_v, addr_v, acc_v, shared):
    cid = lax.axis_index("c")
    sid = lax.axis_index("s")

    pltpu.sync_copy(zeros_hbm, acc_v)

    for cl in range(_CPC):
        for r4 in range(_RPS):
            p = (cid * _CPC + cl) * _NR + sid * _RPS + r4
            pltpu.sync_copy(addr_hbm.at[pl.ds(p * _B, _B)], addr_v)
            pltpu.sync_copy(table_hbm.at[pl.ds(p * _NA, _NA)], row_v)

            def body(i, _, cl=cl):
                iv = addr_v[pl.ds(i * _LANES, _LANES)]
                g = plsc.load_gather(row_v, [iv])
                acc_v[pl.ds(cl * _B + i * _LANES, _LANES)] += g
                return 0

            lax.fori_loop(0, _B // _LANES, body, 0)

    for cl in range(_CPC):
        pltpu.sync_copy(acc_v.at[pl.ds(cl * _B, _B)],
                        shared.at[pl.ds((cl * _NSUB + sid) * _B, _B)])
    plsc.subcore_barrier()

    @pl.when(sid < _CPC)
    def _():
        pltpu.sync_copy(shared.at[pl.ds(sid * (_NSUB * _B), _NSUB * _B)],
                        row_v)

        def rbody(i, _):
            tot = row_v[pl.ds(i * _LANES, _LANES)]
            for k in range(1, _NSUB):
                tot = tot + row_v[pl.ds(k * _B + i * _LANES, _LANES)]
            acc_v[pl.ds(i * _LANES, _LANES)] = tot
            return 0

        lax.fori_loop(0, _B // _LANES, rbody, 0)
        pltpu.sync_copy(acc_v.at[pl.ds(0, _B)],
                        out_hbm.at[pl.ds((cid * _CPC + sid) * _B, _B)])


def kernel(samples, tuple_mapping, ram_table):
    samples_bf = samples.astype(jnp.bfloat16).T
    map2 = tuple_mapping.reshape(_NP, _TUPLE)
    table2 = ram_table.reshape(_NP * _NA)
    addr = _compute_addr(map2, samples_bf)
    zeros = jnp.zeros((_CPC * _B,), jnp.float32)
    out = _sc_lookup(table2, addr.reshape(_NP * _B), zeros)
    return out.reshape(_NC, _B).T.astype(jnp.int8)
